# baseline (speedup 1.0000x reference)
import functools

import jax
import jax.numpy as jnp
from jax import lax
from jax.experimental import pallas as pl
from jax.experimental.pallas import tpu as pltpu
from jax.experimental.pallas import tpu_sc as plsc

NC, NS, L = 2, 16, 16
NW = NC * NS
NCOL = 26
BATCH = 16384
DIM = 64
SEG = 100000
FLAT = BATCH * NCOL
PER_W = FLAT // NW
CH = 128
NCHUNK = PER_W // CH


def _build():
    mesh = plsc.VectorSubcoreMesh(
        core_axis_name="c", subcore_axis_name="s",
        num_cores=NC, num_subcores=NS,
    )

    @functools.partial(
        pl.kernel,
        out_type=jax.ShapeDtypeStruct((FLAT, DIM), jnp.float32),
        mesh=mesh,
        compiler_params=pltpu.CompilerParams(use_tc_tiling_on_sc=False),
        scratch_types=[
            pltpu.VMEM((NCHUNK, CH), jnp.int32),
            pltpu.VMEM((NCHUNK, CH), jnp.int32),
            pltpu.VMEM((CH, DIM), jnp.float32),
            pltpu.SemaphoreType.DMA,
        ],
    )
    def k(x_hbm, table_hbm, offs_hbm, out_hbm, idx_v, offs_v, rows, sem):
        wid = lax.axis_index("c") * NS + lax.axis_index("s")
        base = wid * PER_W

        pltpu.sync_copy(x_hbm.at[wid], idx_v)
        pltpu.sync_copy(offs_hbm, offs_v)

        @pl.loop(0, NCHUNK)
        def _add_offsets(j):
            row_i = idx_v.at[j]
            row_o = offs_v.at[j]
            for kk in range(CH // L):
                sl = pl.ds(kk * L, L)
                row_i[sl] = row_i[sl] + row_o[sl]

        @pl.loop(0, NCHUNK)
        def _move(j):
            pltpu.async_copy(table_hbm.at[idx_v.at[j]], rows, sem).wait()
            pltpu.async_copy(rows, out_hbm.at[pl.ds(base + j * CH, CH)], sem).wait()

    return k


_lookup = _build()


def kernel(x_categ, table):
    offs = jnp.tile(jnp.arange(NCOL, dtype=jnp.int32) * SEG, PER_W // NCOL)
    x_flat = x_categ.astype(jnp.int32).reshape(NW, NCHUNK, CH)
    out = _lookup(x_flat, table, offs.reshape(NCHUNK, CH))
    return out.reshape(BATCH, NCOL, DIM)


# speedup vs baseline: 1.0401x; 1.0401x over previous
import functools

import jax
import jax.numpy as jnp
from jax import lax
from jax.experimental import pallas as pl
from jax.experimental.pallas import tpu as pltpu
from jax.experimental.pallas import tpu_sc as plsc

NC, NS, L = 2, 16, 16
NW = NC * NS
NCOL = 26
BATCH = 16384
DIM = 64
SEG = 100000
FLAT = BATCH * NCOL
PER_W = FLAT // NW
CH = 128
NCHUNK = PER_W // CH
NBUF = 8
K = 4

assert NCHUNK % NBUF == 0


def _build():
    mesh = plsc.VectorSubcoreMesh(
        core_axis_name="c", subcore_axis_name="s",
        num_cores=NC, num_subcores=NS,
    )

    @functools.partial(
        pl.kernel,
        out_type=jax.ShapeDtypeStruct((FLAT, DIM), jnp.float32),
        mesh=mesh,
        compiler_params=pltpu.CompilerParams(use_tc_tiling_on_sc=False),
        scratch_types=[
            pltpu.VMEM((NCHUNK, CH), jnp.int32),
            pltpu.VMEM((NCHUNK, CH), jnp.int32),
            pltpu.VMEM((NBUF, CH, DIM), jnp.float32),
            pltpu.SemaphoreType.DMA((NBUF,)),
            pltpu.SemaphoreType.DMA((NBUF,)),
        ],
    )
    def k(x_hbm, table_hbm, offs_hbm, out_hbm, idx_v, offs_v, rows, gsem, ssem):
        wid = lax.axis_index("c") * NS + lax.axis_index("s")
        base = wid * PER_W

        pltpu.sync_copy(x_hbm.at[wid], idx_v)
        pltpu.sync_copy(offs_hbm, offs_v)

        def add_offsets(j):
            row_i = idx_v.at[j]
            row_o = offs_v.at[j]
            for kk in range(CH // L):
                sl = pl.ds(kk * L, L)
                row_i[sl] = row_i[sl] + row_o[sl]

        def gather(j, b):
            return pltpu.make_async_copy(
                table_hbm.at[idx_v.at[j]], rows.at[b], gsem.at[b])

        def scatter(j, b):
            return pltpu.make_async_copy(
                rows.at[b], out_hbm.at[pl.ds(base + j * CH, CH)], ssem.at[b])

        for j in range(K):
            add_offsets(j)
            gather(j, j % NBUF).start()

        @pl.loop(0, NCHUNK, step=NBUF)
        def _pipe(jj):
            for b in range(NBUF):
                j = jj + b
                gather(j, b).wait()
                scatter(j, b).start()
                j2 = j + K
                b2 = (b + K) % NBUF

                @pl.when(j2 < NCHUNK)
                def _():
                    add_offsets(j2)

                    @pl.when(j2 >= NBUF)
                    def _():
                        scatter(j2 - NBUF, b2).wait()

                    gather(j2, b2).start()

        for j in range(NCHUNK - NBUF, NCHUNK):
            scatter(j, j % NBUF).wait()

    return k


_lookup = _build()


def kernel(x_categ, table):
    offs = jnp.tile(jnp.arange(NCOL, dtype=jnp.int32) * SEG, PER_W // NCOL)
    x_flat = x_categ.astype(jnp.int32).reshape(NW, NCHUNK, CH)
    out = _lookup(x_flat, table, offs.reshape(NCHUNK, CH))
    return out.reshape(BATCH, NCOL, DIM)
